# baseline (speedup 1.0000x reference)
import functools

import jax
import jax.numpy as jnp
from jax import lax
from jax.experimental import pallas as pl
from jax.experimental.pallas import tpu as pltpu
from jax.experimental.pallas import tpu_sc as plsc

NC = 2
NS = 16
NW = NC * NS
LANES = 16


def _matmul_body(x_ref, w_ref, o_ref):
    o_ref[...] = jnp.dot(x_ref[...], w_ref[...],
                         preferred_element_type=jnp.float32)


def _add_body(p_ref, o_ref):
    o_ref[...] = p_ref[0] + p_ref[1]


def _make_sc_spmm(n, d, ngroup, gchunk, chunk):
    zrows = 40
    assert n % zrows == 0 and zrows % 8 == 0
    n_zchunk = n // zrows
    n_zloop = (n_zchunk + NS - 1) // NS
    vregs_per_row = d // LANES

    mesh = plsc.VectorSubcoreMesh(core_axis_name="c", subcore_axis_name="s",
                                  num_cores=NC, num_subcores=NS)

    @functools.partial(
        pl.kernel,
        out_type=jax.ShapeDtypeStruct((NC, n, d), jnp.float32),
        mesh=mesh,
        scratch_types=[
            pltpu.VMEM((gchunk, chunk), jnp.int32),
            pltpu.VMEM((gchunk, chunk), jnp.int32),
            pltpu.VMEM((gchunk, chunk), jnp.float32),
            pltpu.VMEM((chunk, d), jnp.float32),
            pltpu.VMEM_SHARED((n, d), jnp.float32),
            pltpu.SemaphoreType.DMA,
        ],
    )
    def sc_spmm(h_hbm, row_hbm, col_hbm, a_hbm, zeros_hbm, out_hbm,
                row_v, col_v, a_v, gbuf, acc, sem):
        c = lax.axis_index("c")
        s = lax.axis_index("s")
        wid = s * NC + c

        for k in range(n_zloop):
            idx = k * NS + s

            @pl.when(idx < n_zchunk)
            def _():
                pltpu.sync_copy(zeros_hbm, acc.at[pl.ds(idx * zrows, zrows)])
        plsc.subcore_barrier()

        def group_loop(q, carry):
            pltpu.sync_copy(row_hbm.at[wid, q], row_v)
            pltpu.sync_copy(col_hbm.at[wid, q], col_v)
            pltpu.sync_copy(a_hbm.at[wid, q], a_v)

            for cc in range(gchunk):
                pltpu.async_copy(h_hbm.at[col_v.at[cc]], gbuf, sem).wait()

                def scale_body(g, carry2, cc=cc):
                    av16 = a_v[cc, pl.ds(g * LANES, LANES)]
                    for i in range(LANES):
                        ab = jnp.broadcast_to(av16[i], (LANES,))
                        e = g * LANES + i
                        for f in range(vregs_per_row):
                            sl = pl.ds(f * LANES, LANES)
                            gbuf[e, sl] = gbuf[e, sl] * ab
                    return carry2

                lax.fori_loop(0, chunk // LANES, scale_body, 0)
                pltpu.sync_copy(gbuf, acc.at[row_v.at[cc]], add=True)
            return carry

        lax.fori_loop(0, ngroup, group_loop, 0)
        plsc.subcore_barrier()

        for k in range(n_zloop):
            idx = k * NS + s

            @pl.when(idx < n_zchunk)
            def _():
                base = idx * zrows
                pltpu.sync_copy(acc.at[pl.ds(base, zrows)],
                                out_hbm.at[c, pl.ds(base, zrows)])

    return sc_spmm


def kernel(x, edge_index, A_values, W):
    n, d_in = x.shape
    d_out = W.shape[1]
    e = A_values.shape[0]

    chunk = 128
    gchunk = 16
    ngroup = 5
    ew = ngroup * gchunk * chunk
    pad = NW * ew - e
    assert pad >= 0

    blk = 1000
    h = pl.pallas_call(
        _matmul_body,
        grid=(n // blk,),
        in_specs=[
            pl.BlockSpec((blk, d_in), lambda i: (i, 0)),
            pl.BlockSpec((d_in, d_out), lambda i: (0, 0)),
        ],
        out_specs=pl.BlockSpec((blk, d_out), lambda i: (i, 0)),
        out_shape=jax.ShapeDtypeStruct((n, d_out), jnp.float32),
    )(x, W)

    row4 = jnp.pad(edge_index[0], (0, pad)).reshape(NW, ngroup, gchunk, chunk)
    col4 = jnp.pad(edge_index[1], (0, pad)).reshape(NW, ngroup, gchunk, chunk)
    a4 = jnp.pad(A_values, (0, pad)).reshape(NW, ngroup, gchunk, chunk)
    zeros = jnp.zeros((40, d_out), jnp.float32)

    partials = _make_sc_spmm(n, d_out, ngroup, gchunk, chunk)(
        h, row4, col4, a4, zeros)

    out = pl.pallas_call(
        _add_body,
        grid=(n // blk,),
        in_specs=[pl.BlockSpec((NC, blk, d_out), lambda i: (0, i, 0))],
        out_specs=pl.BlockSpec((blk, d_out), lambda i: (i, 0)),
        out_shape=jax.ShapeDtypeStruct((n, d_out), jnp.float32),
    )(partials)
    return out


# speedup vs baseline: 2.1283x; 2.1283x over previous
import functools

import jax
import jax.numpy as jnp
from jax import lax
from jax.experimental import pallas as pl
from jax.experimental.pallas import tpu as pltpu
from jax.experimental.pallas import tpu_sc as plsc

NC = 2
NS = 16
NW = NC * NS
LANES = 16


def _matmul_body(x_ref, w_ref, o_ref):
    o_ref[...] = jnp.dot(x_ref[...], w_ref[...],
                         preferred_element_type=jnp.float32)


def _add_body(p_ref, o_ref):
    o_ref[...] = p_ref[0] + p_ref[1]


def _make_sc_spmm(n, d, ngroup, gchunk, chunk):
    zrows = 40
    assert n % zrows == 0 and zrows % 8 == 0
    n_zchunk = n // zrows
    n_zloop = (n_zchunk + NS - 1) // NS
    vregs_per_row = d // LANES

    mesh = plsc.VectorSubcoreMesh(core_axis_name="c", subcore_axis_name="s",
                                  num_cores=NC, num_subcores=NS)

    @functools.partial(
        pl.kernel,
        out_type=jax.ShapeDtypeStruct((NC, n, d), jnp.float32),
        mesh=mesh,
        scratch_types=[
            pltpu.VMEM((gchunk, chunk), jnp.int32),
            pltpu.VMEM((gchunk, chunk), jnp.int32),
            pltpu.VMEM((gchunk, chunk), jnp.float32),
            pltpu.VMEM((chunk, d), jnp.float32),
            pltpu.VMEM_SHARED((n, d), jnp.float32),
            pltpu.SemaphoreType.DMA,
        ],
    )
    def sc_spmm(h_hbm, row_hbm, col_hbm, a_hbm, zeros_hbm, out_hbm,
                row_v, col_v, a_v, gbuf, acc, sem):
        c = lax.axis_index("c")
        s = lax.axis_index("s")
        wid = s * NC + c

        for k in range(n_zloop):
            idx = k * NS + s

            @pl.when(idx < n_zchunk)
            def _():
                pltpu.sync_copy(zeros_hbm, acc.at[pl.ds(idx * zrows, zrows)])
        plsc.subcore_barrier()

        def group_loop(q, carry):
            pltpu.sync_copy(row_hbm.at[wid, q], row_v)
            pltpu.sync_copy(col_hbm.at[wid, q], col_v)
            pltpu.sync_copy(a_hbm.at[wid, q], a_v)

            for cc in range(gchunk):
                pltpu.async_copy(h_hbm.at[col_v.at[cc]], gbuf, sem).wait()

                def scale_body(g, carry2, cc=cc):
                    av16 = a_v[cc, pl.ds(g * LANES, LANES)]
                    for i in range(LANES):
                        ab = jnp.broadcast_to(av16[i], (LANES,))
                        e = g * LANES + i
                        for f in range(vregs_per_row):
                            sl = pl.ds(f * LANES, LANES)
                            gbuf[e, sl] = gbuf[e, sl] * ab
                    return carry2

                lax.fori_loop(0, chunk // LANES, scale_body, 0)
                pltpu.sync_copy(gbuf, acc.at[row_v.at[cc]], add=True)
            return carry

        lax.fori_loop(0, ngroup, group_loop, 0)
        plsc.subcore_barrier()

        for k in range(n_zloop):
            idx = k * NS + s

            @pl.when(idx < n_zchunk)
            def _():
                base = idx * zrows
                pltpu.sync_copy(acc.at[pl.ds(base, zrows)],
                                out_hbm.at[c, pl.ds(base, zrows)])

    return sc_spmm


def kernel(x, edge_index, A_values, W):
    n, d_in = x.shape
    d_out = W.shape[1]
    e = A_values.shape[0]

    chunk = 128
    gchunk = 16
    ngroup = 5
    ew = ngroup * gchunk * chunk
    pad = NW * ew - e
    assert pad >= 0

    blk = 1000
    h = pl.pallas_call(
        _matmul_body,
        grid=(n // blk,),
        in_specs=[
            pl.BlockSpec((blk, d_in), lambda i: (i, 0)),
            pl.BlockSpec((d_in, d_out), lambda i: (0, 0)),
        ],
        out_specs=pl.BlockSpec((blk, d_out), lambda i: (i, 0)),
        out_shape=jax.ShapeDtypeStruct((n, d_out), jnp.float32),
    )(x, W)

    pidx = (jnp.arange(pad, dtype=jnp.int32) * 37) % n
    row4 = jnp.concatenate([edge_index[0], pidx]).reshape(
        NW, ngroup, gchunk, chunk)
    col4 = jnp.concatenate([edge_index[1], pidx]).reshape(
        NW, ngroup, gchunk, chunk)
    a4 = jnp.pad(A_values, (0, pad)).reshape(NW, ngroup, gchunk, chunk)
    zeros = jnp.zeros((40, d_out), jnp.float32)

    partials = _make_sc_spmm(n, d_out, ngroup, gchunk, chunk)(
        h, row4, col4, a4, zeros)

    out = pl.pallas_call(
        _add_body,
        grid=(n // blk,),
        in_specs=[pl.BlockSpec((NC, blk, d_out), lambda i: (0, i, 0))],
        out_specs=pl.BlockSpec((blk, d_out), lambda i: (i, 0)),
        out_shape=jax.ShapeDtypeStruct((n, d_out), jnp.float32),
    )(partials)
    return out


# speedup vs baseline: 2.4865x; 1.1683x over previous
import functools

import jax
import jax.numpy as jnp
from jax import lax
from jax.experimental import pallas as pl
from jax.experimental.pallas import tpu as pltpu
from jax.experimental.pallas import tpu_sc as plsc

NC = 2
NS = 16
NW = NC * NS
LANES = 16


def _matmul_body(x_ref, w_ref, o_ref):
    o_ref[...] = jnp.dot(x_ref[...], w_ref[...],
                         preferred_element_type=jnp.float32)


def _add_body(p_ref, o_ref):
    o_ref[...] = p_ref[0] + p_ref[1]


def _make_sc_spmm(n, d, ngroup, gchunk, chunk):
    zrows = 40
    assert n % zrows == 0 and zrows % 8 == 0
    n_zchunk = n // zrows
    n_zloop = (n_zchunk + NS - 1) // NS
    vregs_per_row = d // LANES

    mesh = plsc.VectorSubcoreMesh(core_axis_name="c", subcore_axis_name="s",
                                  num_cores=NC, num_subcores=NS)

    @functools.partial(
        pl.kernel,
        out_type=jax.ShapeDtypeStruct((NC, n, d), jnp.float32),
        mesh=mesh,
        scratch_types=[
            pltpu.VMEM((gchunk, chunk), jnp.int32),
            pltpu.VMEM((gchunk, chunk), jnp.int32),
            pltpu.VMEM((gchunk, chunk), jnp.float32),
            pltpu.VMEM((chunk, d), jnp.float32),
            pltpu.VMEM((chunk, d), jnp.float32),
            pltpu.VMEM_SHARED((n, d), jnp.float32),
            pltpu.SemaphoreType.DMA,
            pltpu.SemaphoreType.DMA,
        ],
    )
    def sc_spmm(h_hbm, row_hbm, col_hbm, a_hbm, zeros_hbm, out_hbm,
                row_v, col_v, a_v, gbufA, gbufB, acc, semA, semB):
        c = lax.axis_index("c")
        s = lax.axis_index("s")
        wid = s * NC + c

        for k in range(n_zloop):
            idx = k * NS + s

            @pl.when(idx < n_zchunk)
            def _():
                pltpu.sync_copy(zeros_hbm, acc.at[pl.ds(idx * zrows, zrows)])
        plsc.subcore_barrier()

        bufs = (gbufA, gbufB)
        sems = (semA, semB)

        def group_loop(q, carry):
            pltpu.sync_copy(row_hbm.at[wid, q], row_v)
            pltpu.sync_copy(col_hbm.at[wid, q], col_v)
            pltpu.sync_copy(a_hbm.at[wid, q], a_v)

            desc = pltpu.async_copy(h_hbm.at[col_v.at[0]], bufs[0], sems[0])
            for cc in range(gchunk):
                gbuf = bufs[cc % 2]
                if cc + 1 < gchunk:
                    nxt = pltpu.async_copy(
                        h_hbm.at[col_v.at[cc + 1]], bufs[(cc + 1) % 2],
                        sems[(cc + 1) % 2])
                desc.wait()
                desc = nxt if cc + 1 < gchunk else None

                def scale_body(g, carry2, cc=cc, gbuf=gbuf):
                    av16 = a_v[cc, pl.ds(g * LANES, LANES)]
                    for i in range(LANES):
                        ab = jnp.broadcast_to(av16[i], (LANES,))
                        e = g * LANES + i
                        for f in range(vregs_per_row):
                            sl = pl.ds(f * LANES, LANES)
                            gbuf[e, sl] = gbuf[e, sl] * ab
                    return carry2

                lax.fori_loop(0, chunk // LANES, scale_body, 0)
                pltpu.sync_copy(gbuf, acc.at[row_v.at[cc]], add=True)
            return carry

        lax.fori_loop(0, ngroup, group_loop, 0)
        plsc.subcore_barrier()

        for k in range(n_zloop):
            idx = k * NS + s

            @pl.when(idx < n_zchunk)
            def _():
                base = idx * zrows
                pltpu.sync_copy(acc.at[pl.ds(base, zrows)],
                                out_hbm.at[c, pl.ds(base, zrows)])

    return sc_spmm


def kernel(x, edge_index, A_values, W):
    n, d_in = x.shape
    d_out = W.shape[1]
    e = A_values.shape[0]

    chunk = 80
    gchunk = 8
    ngroup = 16
    ew = ngroup * gchunk * chunk
    pad = NW * ew - e
    assert pad >= 0

    blk = 1000
    h = pl.pallas_call(
        _matmul_body,
        grid=(n // blk,),
        in_specs=[
            pl.BlockSpec((blk, d_in), lambda i: (i, 0)),
            pl.BlockSpec((d_in, d_out), lambda i: (0, 0)),
        ],
        out_specs=pl.BlockSpec((blk, d_out), lambda i: (i, 0)),
        out_shape=jax.ShapeDtypeStruct((n, d_out), jnp.float32),
    )(x, W)

    pidx = (jnp.arange(pad, dtype=jnp.int32) * 37) % n
    row4 = jnp.concatenate([edge_index[0], pidx]).reshape(
        NW, ngroup, gchunk, chunk)
    col4 = jnp.concatenate([edge_index[1], pidx]).reshape(
        NW, ngroup, gchunk, chunk)
    a4 = jnp.pad(A_values, (0, pad)).reshape(NW, ngroup, gchunk, chunk)
    zeros = jnp.zeros((40, d_out), jnp.float32)

    partials = _make_sc_spmm(n, d_out, ngroup, gchunk, chunk)(
        h, row4, col4, a4, zeros)

    out = pl.pallas_call(
        _add_body,
        grid=(n // blk,),
        in_specs=[pl.BlockSpec((NC, blk, d_out), lambda i: (0, i, 0))],
        out_specs=pl.BlockSpec((blk, d_out), lambda i: (i, 0)),
        out_shape=jax.ShapeDtypeStruct((n, d_out), jnp.float32),
    )(partials)
    return out
